# Initial kernel scaffold; baseline (speedup 1.0000x reference)
import functools

import jax
import jax.numpy as jnp
from jax.experimental import pallas as pl
from jax.experimental.pallas import tpu as pltpu
from jax.experimental.pallas import tpu_sc as plsc

N = 10000
D = 128
E = 320000
K_HOPS = 2
NC = 2
NS = 16
EPW = E // (NC * NS)
CHUNK = 400
NCHUNK = EPW // CHUNK
ROWS_PER_SUB = 1000
DEG_W = 16


def _vector_mesh():
    return plsc.VectorSubcoreMesh(core_axis_name="c", subcore_axis_name="s")


def _deg_sc(dst, zeros_nw, ones_cw):
    @functools.partial(
        pl.kernel,
        out_type=jax.ShapeDtypeStruct((NC, N, DEG_W), jnp.float32),
        mesh=_vector_mesh(),
        scratch_types=[
            pltpu.VMEM((CHUNK,), jnp.int32),
            pltpu.VMEM((CHUNK, DEG_W), jnp.float32),
            pltpu.VMEM_SHARED((N, DEG_W), jnp.float32),
        ],
    )
    def k(dst_hbm, z_hbm, ones_hbm, out_hbm, didx, ones_v, acc):
        cid = jax.lax.axis_index("c")
        sid = jax.lax.axis_index("s")

        @pl.when(sid < N // ROWS_PER_SUB)
        def _():
            r = sid * ROWS_PER_SUB
            pltpu.sync_copy(
                z_hbm.at[pl.ds(r, ROWS_PER_SUB)], acc.at[pl.ds(r, ROWS_PER_SUB)]
            )

        pltpu.sync_copy(ones_hbm, ones_v)
        plsc.subcore_barrier()

        base0 = (cid * NS + sid) * EPW

        @pl.loop(0, NCHUNK)
        def _(c):
            b = base0 + c * CHUNK
            pltpu.sync_copy(dst_hbm.at[pl.ds(b, CHUNK)], didx)
            pltpu.sync_copy(ones_v, acc.at[didx], add=True)

        plsc.subcore_barrier()

        @pl.when(sid < N // ROWS_PER_SUB)
        def _():
            r = sid * ROWS_PER_SUB
            pltpu.sync_copy(
                acc.at[pl.ds(r, ROWS_PER_SUB)],
                out_hbm.at[cid, pl.ds(r, ROWS_PER_SUB)],
            )

    return k(dst, zeros_nw, ones_cw)


def _hop_sc(h, src, dst, zeros_nd):
    @functools.partial(
        pl.kernel,
        out_type=jax.ShapeDtypeStruct((NC, N, D), jnp.float32),
        mesh=_vector_mesh(),
        scratch_types=[
            pltpu.VMEM((CHUNK,), jnp.int32),
            pltpu.VMEM((CHUNK,), jnp.int32),
            pltpu.VMEM((CHUNK, D), jnp.float32),
            pltpu.VMEM_SHARED((N, D), jnp.float32),
            pltpu.SemaphoreType.DMA,
        ],
    )
    def k(h_hbm, src_hbm, dst_hbm, z_hbm, out_hbm, sidx, didx, rows, acc, sem):
        cid = jax.lax.axis_index("c")
        sid = jax.lax.axis_index("s")

        @pl.when(sid < N // ROWS_PER_SUB)
        def _():
            r = sid * ROWS_PER_SUB
            pltpu.sync_copy(
                z_hbm.at[pl.ds(r, ROWS_PER_SUB)], acc.at[pl.ds(r, ROWS_PER_SUB)]
            )

        plsc.subcore_barrier()

        base0 = (cid * NS + sid) * EPW

        @pl.loop(0, NCHUNK)
        def _(c):
            b = base0 + c * CHUNK
            pltpu.sync_copy(src_hbm.at[pl.ds(b, CHUNK)], sidx)
            pltpu.async_copy(h_hbm.at[sidx], rows, sem).wait()
            pltpu.sync_copy(dst_hbm.at[pl.ds(b, CHUNK)], didx)
            pltpu.sync_copy(rows, acc.at[didx], add=True)

        plsc.subcore_barrier()

        @pl.when(sid < N // ROWS_PER_SUB)
        def _():
            r = sid * ROWS_PER_SUB
            pltpu.sync_copy(
                acc.at[pl.ds(r, ROWS_PER_SUB)],
                out_hbm.at[cid, pl.ds(r, ROWS_PER_SUB)],
            )

    return k(h, src, dst, zeros_nd)


def _prep_tc(node_feat, degp):
    def body(nf_ref, degp_ref, hs_ref, dn_ref):
        deg = degp_ref[0, :, 0:1] + degp_ref[1, :, 0:1]
        dn = jnp.where(deg > 0, jax.lax.rsqrt(jnp.maximum(deg, 1.0)), 0.0)
        dn_ref[...] = dn
        hs_ref[...] = nf_ref[...] * dn

    return pl.pallas_call(
        body,
        out_shape=(
            jax.ShapeDtypeStruct((N, D), jnp.float32),
            jax.ShapeDtypeStruct((N, 1), jnp.float32),
        ),
    )(node_feat, degp)


def _standardize(t):
    mu = jnp.mean(t, axis=0, keepdims=True)
    c = t - mu
    sd = jnp.sqrt(jnp.sum(c * c, axis=0, keepdims=True) / (N - 1))
    return c / (sd + 1e-5)


def _mid_tc(p, dn):
    def body(p_ref, dn_ref, out_ref):
        t = (p_ref[0] + p_ref[1]) * dn_ref[...]
        out_ref[...] = _standardize(t) * dn_ref[...]

    return pl.pallas_call(
        body, out_shape=jax.ShapeDtypeStruct((N, D), jnp.float32)
    )(p, dn)


def _final_tc(p, dn, W, b2):
    def body(p_ref, dn_ref, w_ref, b_ref, out_ref):
        t = (p_ref[0] + p_ref[1]) * dn_ref[...]
        t = _standardize(t)
        out_ref[...] = (
            jnp.dot(t, w_ref[...], preferred_element_type=jnp.float32)
            + b_ref[...]
        )

    return pl.pallas_call(
        body, out_shape=jax.ShapeDtypeStruct((N, D), jnp.float32)
    )(p, dn, W, b2)


def kernel(node_feat, edge_index, W, b):
    src = edge_index[0]
    dst = edge_index[1]
    zeros_nd = jnp.zeros((N, D), jnp.float32)
    zeros_nw = jnp.zeros((N, DEG_W), jnp.float32)
    ones_cw = jnp.ones((CHUNK, DEG_W), jnp.float32)

    degp = _deg_sc(dst, zeros_nw, ones_cw)
    hs, dn = _prep_tc(node_feat, degp)
    p = None
    for hop in range(K_HOPS):
        p = _hop_sc(hs, src, dst, zeros_nd)
        if hop < K_HOPS - 1:
            hs = _mid_tc(p, dn)
    return _final_tc(p, dn, W, b.reshape(1, D))


# speedup vs baseline: 6.7619x; 6.7619x over previous
import functools

import jax
import jax.numpy as jnp
from jax.experimental import pallas as pl
from jax.experimental.pallas import tpu as pltpu
from jax.experimental.pallas import tpu_sc as plsc

N = 10000
D = 128
E = 320000
K_HOPS = 2
NC = 2
NS = 16
EPW = E // (NC * NS)
CHUNK = 200
NCHUNK = EPW // CHUNK
ROWS_PER_SUB = 1000
DEG_W = 128


def _vector_mesh():
    return plsc.VectorSubcoreMesh(core_axis_name="c", subcore_axis_name="s")


def _deg_sc(dst, zeros_nw, ones_cw):
    @functools.partial(
        pl.kernel,
        out_type=jax.ShapeDtypeStruct((NC, N, DEG_W), jnp.float32),
        mesh=_vector_mesh(),
        scratch_types=[
            pltpu.VMEM((CHUNK,), jnp.int32),
            pltpu.VMEM((CHUNK, DEG_W), jnp.float32),
            pltpu.VMEM_SHARED((N, DEG_W), jnp.float32),
        ],
    )
    def k(dst_hbm, z_hbm, ones_hbm, out_hbm, didx, ones_v, acc):
        cid = jax.lax.axis_index("c")
        sid = jax.lax.axis_index("s")

        @pl.when(sid < N // ROWS_PER_SUB)
        def _():
            r = sid * ROWS_PER_SUB
            pltpu.sync_copy(
                z_hbm.at[pl.ds(r, ROWS_PER_SUB)], acc.at[pl.ds(r, ROWS_PER_SUB)]
            )

        pltpu.sync_copy(ones_hbm, ones_v)
        plsc.subcore_barrier()

        base0 = (cid * NS + sid) * EPW

        @pl.loop(0, NCHUNK)
        def _(c):
            b = base0 + c * CHUNK
            pltpu.sync_copy(dst_hbm.at[pl.ds(b, CHUNK)], didx)
            pltpu.sync_copy(ones_v, acc.at[didx], add=True)

        plsc.subcore_barrier()

        @pl.when(sid < N // ROWS_PER_SUB)
        def _():
            r = sid * ROWS_PER_SUB
            pltpu.sync_copy(
                acc.at[pl.ds(r, ROWS_PER_SUB)],
                out_hbm.at[cid, pl.ds(r, ROWS_PER_SUB)],
            )

    return k(dst, zeros_nw, ones_cw)


def _hop_sc(h, src, dst, zeros_nd):
    @functools.partial(
        pl.kernel,
        out_type=jax.ShapeDtypeStruct((NC, N, D), jnp.float32),
        mesh=_vector_mesh(),
        scratch_types=[
            pltpu.VMEM((CHUNK,), jnp.int32),
            pltpu.VMEM((CHUNK,), jnp.int32),
            pltpu.VMEM((CHUNK, D), jnp.float32),
            pltpu.VMEM_SHARED((N, D), jnp.float32),
            pltpu.SemaphoreType.DMA,
        ],
    )
    def k(h_hbm, src_hbm, dst_hbm, z_hbm, out_hbm, sidx, didx, rows, acc, sem):
        cid = jax.lax.axis_index("c")
        sid = jax.lax.axis_index("s")

        @pl.when(sid < N // ROWS_PER_SUB)
        def _():
            r = sid * ROWS_PER_SUB
            pltpu.sync_copy(
                z_hbm.at[pl.ds(r, ROWS_PER_SUB)], acc.at[pl.ds(r, ROWS_PER_SUB)]
            )

        plsc.subcore_barrier()

        base0 = (cid * NS + sid) * EPW

        @pl.loop(0, NCHUNK)
        def _(c):
            b = base0 + c * CHUNK
            pltpu.sync_copy(src_hbm.at[pl.ds(b, CHUNK)], sidx)
            pltpu.async_copy(h_hbm.at[sidx], rows, sem).wait()
            pltpu.sync_copy(dst_hbm.at[pl.ds(b, CHUNK)], didx)
            pltpu.sync_copy(rows, acc.at[didx], add=True)

        plsc.subcore_barrier()

        @pl.when(sid < N // ROWS_PER_SUB)
        def _():
            r = sid * ROWS_PER_SUB
            pltpu.sync_copy(
                acc.at[pl.ds(r, ROWS_PER_SUB)],
                out_hbm.at[cid, pl.ds(r, ROWS_PER_SUB)],
            )

    return k(h, src, dst, zeros_nd)


def _prep_tc(node_feat, degp):
    def body(nf_ref, degp_ref, hs_ref, dn_ref):
        deg = degp_ref[0, :, 0:1] + degp_ref[1, :, 0:1]
        dn = jnp.where(deg > 0, jax.lax.rsqrt(jnp.maximum(deg, 1.0)), 0.0)
        dn_ref[...] = dn
        hs_ref[...] = nf_ref[...] * dn

    return pl.pallas_call(
        body,
        out_shape=(
            jax.ShapeDtypeStruct((N, D), jnp.float32),
            jax.ShapeDtypeStruct((N, 1), jnp.float32),
        ),
    )(node_feat, degp)


def _standardize(t):
    mu = jnp.mean(t, axis=0, keepdims=True)
    c = t - mu
    sd = jnp.sqrt(jnp.sum(c * c, axis=0, keepdims=True) / (N - 1))
    return c / (sd + 1e-5)


def _mid_tc(p, dn):
    def body(p_ref, dn_ref, out_ref):
        t = (p_ref[0] + p_ref[1]) * dn_ref[...]
        out_ref[...] = _standardize(t) * dn_ref[...]

    return pl.pallas_call(
        body, out_shape=jax.ShapeDtypeStruct((N, D), jnp.float32)
    )(p, dn)


def _final_tc(p, dn, W, b2):
    def body(p_ref, dn_ref, w_ref, b_ref, out_ref):
        t = (p_ref[0] + p_ref[1]) * dn_ref[...]
        t = _standardize(t)
        out_ref[...] = (
            jnp.dot(t, w_ref[...], preferred_element_type=jnp.float32)
            + b_ref[...]
        )

    return pl.pallas_call(
        body, out_shape=jax.ShapeDtypeStruct((N, D), jnp.float32)
    )(p, dn, W, b2)


def kernel(node_feat, edge_index, W, b):
    src = edge_index[0]
    dst = edge_index[1]
    zeros_nd = jnp.zeros((N, D), jnp.float32)
    zeros_nw = jnp.zeros((N, DEG_W), jnp.float32)
    ones_cw = jnp.ones((CHUNK, DEG_W), jnp.float32)

    degp = _deg_sc(dst, zeros_nw, ones_cw)
    hs, dn = _prep_tc(node_feat, degp)
    p = None
    for hop in range(K_HOPS):
        p = _hop_sc(hs, src, dst, zeros_nd)
        if hop < K_HOPS - 1:
            hs = _mid_tc(p, dn)
    return _final_tc(p, dn, W, b.reshape(1, D))
